# baseline (speedup 1.0000x reference)
import jax
import jax.numpy as jnp
from jax.experimental import pallas as pl
from jax.experimental.pallas import tpu as pltpu


def _router_kernel(x_ref, w_ref, s1_ref, s2_ref, i1_ref, i2_ref):
    x = x_ref[...]
    w = w_ref[...]
    logits = jax.lax.dot_general(
        x, w, (((1,), (1,)), ((), ())),
        preferred_element_type=jnp.float32)
    e = logits.shape[-1]
    m = jnp.max(logits, axis=-1, keepdims=True)
    z = jnp.sum(jnp.exp(logits - m), axis=-1)
    iota = jax.lax.broadcasted_iota(jnp.int32, logits.shape, 1)
    big = jnp.int32(e)
    idx1 = jnp.min(jnp.where(logits == m, iota, big), axis=-1, keepdims=True)
    masked = jnp.where(iota == idx1, -jnp.inf, logits)
    m2 = jnp.max(masked, axis=-1, keepdims=True)
    idx2 = jnp.min(jnp.where(masked == m2, iota, big), axis=-1)
    s1_ref[...] = 1.0 / z
    s2_ref[...] = jnp.exp(m2[:, 0] - m[:, 0]) / z
    i1_ref[...] = idx1[:, 0]
    i2_ref[...] = idx2


def kernel(hidden_states, W):
    B, S, D = hidden_states.shape
    E = W.shape[0]
    N = B * S
    x = hidden_states.reshape(N, D)
    T = 2048
    s1, s2, i1, i2 = pl.pallas_call(
        _router_kernel,
        grid=(N // T,),
        compiler_params=pltpu.CompilerParams(
            dimension_semantics=("arbitrary",)),
        in_specs=[
            pl.BlockSpec((T, D), lambda i: (i, 0)),
            pl.BlockSpec((E, D), lambda i: (0, 0)),
        ],
        out_specs=[
            pl.BlockSpec((T,), lambda i: (i,)),
            pl.BlockSpec((T,), lambda i: (i,)),
            pl.BlockSpec((T,), lambda i: (i,)),
            pl.BlockSpec((T,), lambda i: (i,)),
        ],
        out_shape=[
            jax.ShapeDtypeStruct((N,), jnp.float32),
            jax.ShapeDtypeStruct((N,), jnp.float32),
            jax.ShapeDtypeStruct((N,), jnp.int32),
            jax.ShapeDtypeStruct((N,), jnp.int32),
        ],
    )(x, W)
    scores = jnp.stack([s1, s2], axis=-1).reshape(B, S, 2)
    indices = jnp.stack([i1, i2], axis=-1).reshape(B, S, 2)
    return scores, indices


# speedup vs baseline: 1.5098x; 1.5098x over previous
import jax
import jax.numpy as jnp
from jax.experimental import pallas as pl
from jax.experimental.pallas import tpu as pltpu

_G = 8


def _router_kernel(x_ref, w_ref, s_ref, i_ref, l_ref):
    step = pl.program_id(0)

    @pl.when(step > 0)
    def _epilogue():
        logits = l_ref[...]
        e = logits.shape[-1]
        m = jnp.max(logits, axis=-1, keepdims=True)
        z = jnp.sum(jnp.exp(logits - m), axis=-1, keepdims=True)
        iota = jax.lax.broadcasted_iota(jnp.int32, logits.shape, 1)
        big = jnp.int32(e)
        idx1 = jnp.min(jnp.where(logits == m, iota, big), axis=-1,
                       keepdims=True)
        masked = jnp.where(iota == idx1, -jnp.inf, logits)
        m2 = jnp.max(masked, axis=-1, keepdims=True)
        idx2 = jnp.min(jnp.where(masked == m2, iota, big), axis=-1,
                       keepdims=True)
        s1 = 1.0 / z
        s2 = jnp.exp(m2 - m) / z
        s_ref[...] = jnp.concatenate([s1, s2], axis=-1)
        i_ref[...] = jnp.concatenate([idx1, idx2], axis=-1)

    @pl.when(step < _G)
    def _matmul():
        l_ref[...] = jax.lax.dot_general(
            x_ref[...], w_ref[...], (((1,), (1,)), ((), ())),
            preferred_element_type=jnp.float32)


def kernel(hidden_states, W):
    B, S, D = hidden_states.shape
    E = W.shape[0]
    N = B * S
    x = hidden_states.reshape(N, D)
    T = N // _G
    scores, indices = pl.pallas_call(
        _router_kernel,
        grid=(_G + 1,),
        compiler_params=pltpu.CompilerParams(
            dimension_semantics=("arbitrary",)),
        in_specs=[
            pl.BlockSpec((T, D), lambda i: (jnp.minimum(i, _G - 1), 0)),
            pl.BlockSpec((E, D), lambda i: (0, 0)),
        ],
        out_specs=[
            pl.BlockSpec((T, 2), lambda i: (jnp.maximum(i - 1, 0), 0)),
            pl.BlockSpec((T, 2), lambda i: (jnp.maximum(i - 1, 0), 0)),
        ],
        out_shape=[
            jax.ShapeDtypeStruct((N, 2), jnp.float32),
            jax.ShapeDtypeStruct((N, 2), jnp.int32),
        ],
        scratch_shapes=[pltpu.VMEM((T, E), jnp.float32)],
    )(x, W)
    return scores.reshape(B, S, 2), indices.reshape(B, S, 2)
